# baseline (speedup 1.0000x reference)
import functools

import jax
import jax.numpy as jnp
from jax import lax
from jax.experimental import pallas as pl
from jax.experimental.pallas import tpu as pltpu
from jax.experimental.pallas import tpu_sc as plsc

B = 4
N = 512
C = 16
D = 1024
NBITS = 6
R = B * N
ROW = C * D

NW = 32
RPW = R // NW
ROWB = 2
NBUF = 2
NSTEP = RPW // ROWB
NGRP = D // 16


def _router_body(x_hbm, ht_hbm, eidx_hbm, gates_hbm, conf_hbm,
                 xbuf, htbuf, eidx_v, gates_v, conf_v, sems):
    cid = lax.axis_index("c")
    sid = lax.axis_index("s")
    wid = sid * 2 + cid
    base = wid * RPW

    pltpu.sync_copy(ht_hbm, htbuf)

    ones16 = jnp.full((16,), 1.0, dtype=jnp.float32)
    for i in range(RPW * 2 // 16):
        gates_v[pl.ds(i * 16, 16)] = ones16

    lane = jnp.arange(16, dtype=jnp.int32)
    mask2 = lane < 2
    mask1 = lane < 1

    def issue(buf, step):
        pltpu.make_async_copy(
            x_hbm.at[pl.ds((base + step * ROWB) * C, ROWB * C)],
            xbuf.at[buf], sems.at[buf]).start()

    def wait(buf):
        pltpu.make_async_copy(
            x_hbm.at[pl.ds(0, ROWB * C)], xbuf.at[buf], sems.at[buf]).wait()

    for bf in range(NBUF):
        issue(bf, bf)

    def compute(buf, step):
        def gbody(g, paccs):
            col = g * 16
            hts = [htbuf[j, pl.ds(col, 16)] for j in range(NBITS)]
            out = []
            for r in range(ROWB):
                acc = xbuf[buf, r * C, pl.ds(col, 16)]
                for c in range(1, C):
                    acc = acc + xbuf[buf, r * C + c, pl.ds(col, 16)]
                u = plsc.bitcast(acc * (1.0 / C), jnp.int32)
                u = (u + jnp.int32(0x7FFF) +
                     (lax.shift_right_logical(u, jnp.int32(16)) &
                      jnp.int32(1))) & jnp.int32(-65536)
                emb = plsc.bitcast(u, jnp.float32)
                for j in range(NBITS):
                    out.append(paccs[r * NBITS + j] + emb * hts[j])
            return tuple(out)

        z = jnp.zeros((16,), dtype=jnp.float32)
        paccs = lax.fori_loop(0, NGRP, gbody, (z,) * (ROWB * NBITS))

        for r in range(ROWB):
            ps = [jnp.sum(paccs[r * NBITS + j]) for j in range(NBITS)]
            eid = jnp.int32(0)
            for j in range(NBITS):
                eid = eid + jnp.where(ps[j] > 0, jnp.int32(1 << j),
                                      jnp.int32(0))
            aj = [jnp.abs(p) for p in ps]
            m = aj[0]
            flip = jnp.int32(1)
            for j in range(1, NBITS):
                c = aj[j] < m
                m = jnp.where(c, aj[j], m)
                flip = jnp.where(c, jnp.int32(1 << j), flip)
            e2 = lax.bitwise_xor(eid, flip)
            conf = (aj[0] + aj[1] + aj[2] + aj[3] + aj[4] + aj[5]) \
                * (1.0 / NBITS)

            rl = step * ROWB + r
            ev = jnp.where(lane == 0, jnp.full((16,), eid, jnp.int32),
                           jnp.full((16,), e2, jnp.int32))
            plsc.store_scatter(eidx_v, [2 * rl + lane], ev, mask=mask2)
            plsc.store_scatter(conf_v, [rl + lane * 0],
                               jnp.full((16,), conf, jnp.float32),
                               mask=mask1)

    def sbody(s2, carry):
        for bf in range(NBUF):
            step = s2 * NBUF + bf
            wait(bf)
            compute(bf, step)
            nxt = step + NBUF

            @pl.when(nxt < NSTEP)
            def _():
                issue(bf, nxt)
        return carry

    lax.fori_loop(0, NSTEP // NBUF, sbody, jnp.int32(0))

    pltpu.sync_copy(eidx_v, eidx_hbm.at[pl.ds(base * 2, RPW * 2)])
    pltpu.sync_copy(gates_v, gates_hbm.at[pl.ds(base * 2, RPW * 2)])
    pltpu.sync_copy(conf_v, conf_hbm.at[pl.ds(base, RPW)])


@jax.jit
def _router(x2, ht):
    mesh = plsc.VectorSubcoreMesh(core_axis_name="c", subcore_axis_name="s")
    return pl.kernel(
        _router_body,
        out_type=[
            jax.ShapeDtypeStruct((R * 2,), jnp.int32),
            jax.ShapeDtypeStruct((R * 2,), jnp.float32),
            jax.ShapeDtypeStruct((R,), jnp.float32),
        ],
        mesh=mesh,
        scratch_types=[
            pltpu.VMEM((NBUF, ROWB * C, D), jnp.float32),
            pltpu.VMEM((NBITS, D), jnp.float32),
            pltpu.VMEM((RPW * 2,), jnp.int32),
            pltpu.VMEM((RPW * 2,), jnp.float32),
            pltpu.VMEM((RPW,), jnp.float32),
            pltpu.SemaphoreType.DMA((NBUF,)),
        ],
        compiler_params=pltpu.CompilerParams(needs_layout_passes=False),
    )(x2, ht)


def _round_bf16(v):
    u = lax.bitcast_convert_type(v, jnp.int32)
    u = (u + jnp.int32(0x7FFF) +
         (lax.shift_right_logical(u, 16) & jnp.int32(1))) & jnp.int32(-65536)
    return lax.bitcast_convert_type(u, jnp.float32)


def kernel(x, hyperplanes):
    x2 = x.reshape(R * C, D)
    ht = _round_bf16(hyperplanes.T.reshape(NBITS, D))
    eidx, gates, conf = _router(x2, ht)
    return (eidx.reshape(B, N, 2),
            gates.reshape(B, N, 2),
            conf.reshape(B, N))


# speedup vs baseline: 1.6085x; 1.6085x over previous
import jax, jax.numpy as jnp
from kernel_tc import tc_router

def kernel(x, hyperplanes):
    x2 = x.reshape(32768, 1024)
    hpad = jnp.zeros((1024, 128), jnp.float32).at[:, :6].set(hyperplanes)
    e1, e2, conf = tc_router(x2, hpad, 0, 2048)
    ei = jnp.stack([e1, e2], axis=-1)
    return (ei.reshape(4, 512, 2),
            jnp.ones((4, 512, 2), x.dtype),
            conf.reshape(4, 512))
